# baseline (speedup 1.0000x reference)
import functools

import jax
import jax.numpy as jnp
from jax import lax
from jax.experimental import pallas as pl
from jax.experimental.pallas import tpu as pltpu
from jax.experimental.pallas import tpu_sc as plsc

_NUM_CORES = 2
_NUM_SUBCORES = 16
_CHUNK = 128


@functools.lru_cache(maxsize=None)
def _make_gather(B, V, D):
    nw = _NUM_CORES * _NUM_SUBCORES
    bpw = B // nw
    nch = bpw // _CHUNK
    mesh = plsc.VectorSubcoreMesh(
        core_axis_name="c", subcore_axis_name="s",
        num_cores=_NUM_CORES, num_subcores=_NUM_SUBCORES)

    @functools.partial(
        pl.kernel,
        out_type=jax.ShapeDtypeStruct((B, D), jnp.float32),
        mesh=mesh,
        scratch_types=[
            pltpu.VMEM((bpw,), jnp.int32),
            pltpu.VMEM((bpw, D), jnp.float32),
            pltpu.SemaphoreType.DMA,
        ],
        compiler_params=pltpu.CompilerParams(use_tc_tiling_on_sc=False),
    )
    def gather(labels_hbm, table_hbm, out_hbm, idx_v, rows_v, sem):
        wid = lax.axis_index("s") * _NUM_CORES + lax.axis_index("c")
        base = wid * bpw
        pltpu.sync_copy(labels_hbm.at[pl.ds(base, bpw)], idx_v)
        copies = []
        for j in range(nch):
            c = pltpu.make_async_copy(
                table_hbm.at[idx_v.at[pl.ds(j * _CHUNK, _CHUNK)]],
                rows_v.at[pl.ds(j * _CHUNK, _CHUNK)],
                sem)
            c.start()
            copies.append(c)
        for c in copies:
            c.wait()
        pltpu.sync_copy(rows_v, out_hbm.at[pl.ds(base, bpw)])

    return gather


@jax.jit
def _embed(labels, table):
    (B,) = labels.shape
    V, D = table.shape
    return _make_gather(B, V, D)(labels, table)


def kernel(labels, train, table):
    return _embed(labels.astype(jnp.int32), table)


# speedup vs baseline: 1.7275x; 1.7275x over previous
import functools

import jax
import jax.numpy as jnp
from jax import lax
from jax.experimental import pallas as pl
from jax.experimental.pallas import tpu as pltpu
from jax.experimental.pallas import tpu_sc as plsc

_NUM_CORES = 2
_NUM_SUBCORES = 16


@functools.lru_cache(maxsize=None)
def _make_gather(B, V, D):
    nw = _NUM_CORES * _NUM_SUBCORES
    bpw = B // nw
    mesh = plsc.VectorSubcoreMesh(
        core_axis_name="c", subcore_axis_name="s",
        num_cores=_NUM_CORES, num_subcores=_NUM_SUBCORES)

    @functools.partial(
        pl.kernel,
        out_type=jax.ShapeDtypeStruct((B, D), jnp.float32),
        mesh=mesh,
        scratch_types=[
            pltpu.VMEM((bpw,), jnp.int32),
            pltpu.VMEM((bpw, D), jnp.float32),
            pltpu.SemaphoreType.DMA,
        ],
    )
    def gather(labels_hbm, table_hbm, out_hbm, idx_v, rows_v, sem):
        wid = lax.axis_index("s") * _NUM_CORES + lax.axis_index("c")
        base = wid * bpw
        pltpu.sync_copy(labels_hbm.at[pl.ds(base, bpw)], idx_v)

        def fire(g, carry):
            vec = idx_v[pl.ds(g * 16, 16)]
            for k in range(16):
                pltpu.make_async_copy(
                    table_hbm.at[pl.ds(vec[k], 1)],
                    rows_v.at[pl.ds(g * 16 + k, 1)],
                    sem).start()
            return carry

        lax.fori_loop(0, bpw // 16, fire, 0)
        pltpu.make_async_copy(
            table_hbm.at[pl.ds(0, bpw)], rows_v, sem).wait()
        pltpu.sync_copy(rows_v, out_hbm.at[pl.ds(base, bpw)])

    return gather


@jax.jit
def _embed(labels, table):
    (B,) = labels.shape
    V, D = table.shape
    return _make_gather(B, V, D)(labels, table)


def kernel(labels, train, table):
    return _embed(labels.astype(jnp.int32), table)
